# baseline (speedup 1.0000x reference)
import jax
import jax.numpy as jnp
from jax import lax
from jax.experimental import pallas as pl
from jax.experimental.pallas import tpu as pltpu
from jax.experimental.pallas import tpu_sc as plsc

N_NODES = 100000
N_EDGES = 6400000
R_MAX = 4.0

NC = 2
NS = 16
NW = NC * NS
CHUNK = 2048
N_CHUNKS = N_EDGES // CHUNK
BIG = 21
CPW_HI = 98
NBUF = 3
UNROLL = 8
VEC_ITERS = CHUNK // (16 * UNROLL)

PACK_WORDS = N_NODES // 16
PACK_PAD = 6256

OUT_PAD = 100352
OUT_ROWS = OUT_PAD // 128

CA = 28.0 / 4096.0
CB = 48.0 / 16384.0
CC = 21.0 / 65536.0
RCP_MAGIC = 0x7EF311C3


def _sc_body(len_hbm, eidx_hbm, packed_hbm, sc13_hbm, out_hbm,
             acc, packed_v, sc13_v, len_bufs, idx_bufs, sems):
    cid = lax.axis_index("c")
    sid = lax.axis_index("s")
    wid = sid * NC + cid

    pltpu.sync_copy(packed_hbm, packed_v)
    pltpu.sync_copy(sc13_hbm, sc13_v)

    zeros = jnp.zeros((16,), jnp.float32)

    def zinit(i, _):
        for u in range(10):
            acc[pl.ds(i * 160 + u * 16, 16)] = zeros
        return 0

    lax.fori_loop(0, N_NODES // 160, zinit, 0)

    cnt = jnp.where(wid < BIG, CPW_HI, CPW_HI - 1)
    first = wid * (CPW_HI - 1) + jnp.minimum(wid, BIG)

    def start_chunk(j, buf):
        col = (first + j) * CHUNK
        pltpu.async_copy(len_hbm.at[pl.ds(col, CHUNK)], len_bufs[buf],
                         sems.at[buf])
        pltpu.async_copy(eidx_hbm.at[:, pl.ds(col, CHUNK)], idx_bufs[buf],
                         sems.at[buf])

    def wait_chunk(j, buf):
        col = (first + j) * CHUNK
        pltpu.make_async_copy(len_hbm.at[pl.ds(col, CHUNK)], len_bufs[buf],
                              sems.at[buf]).wait()
        pltpu.make_async_copy(eidx_hbm.at[:, pl.ds(col, CHUNK)],
                              idx_bufs[buf], sems.at[buf]).wait()

    for u in range(NBUF):
        @pl.when(u < cnt)
        def _(u=u):
            start_chunk(u, u)

    def compute_chunk(buf):
        len_v = len_bufs[buf]
        idx_v = idx_bufs[buf]

        @plsc.parallel_loop(0, CHUNK // 16, 1, unroll=UNROLL)
        def vec_body(vi):
            if True:
                off = vi * 16
                ln = len_v[pl.ds(off, 16)]
                c = idx_v[0, pl.ds(off, 16)]
                n = idx_v[1, pl.ds(off, 16)]
                wc = plsc.load_gather(packed_v, [c >> 4])
                wn = plsc.load_gather(packed_v, [n >> 4])
                spc = (wc >> ((c & 15) * 2)) & 3
                spn = (wn >> ((n & 15) * 2)) & 3
                l13 = plsc.load_gather(sc13_v, [spc * 4 + spn])
                y = 1.0 / ln
                yy = y * y
                y4 = yy * yy
                y8 = y4 * y4
                inner = y8 - CA * yy + (CB * y - CC)
                eng = jnp.where(ln < R_MAX, y4 * inner * l13, 0.0)
                plsc.addupdate_scatter(acc, [c], eng)

    def tri_body(i, _):
        for u in range(NBUF):
            j = i * NBUF + u

            @pl.when(j < cnt)
            def _(j=j, u=u):
                wait_chunk(j, u)
                compute_chunk(u)

            @pl.when(j + NBUF < cnt)
            def _(j=j, u=u):
                start_chunk(j + NBUF, u)
        return 0

    lax.fori_loop(0, (CPW_HI + NBUF - 1) // NBUF, tri_body, 0)

    pltpu.sync_copy(acc, out_hbm.at[pl.ds(wid * OUT_PAD, N_NODES)])


@jax.jit
def _sc_edge_partials(edge_length, edge_index, packed, sc13):
    mesh = plsc.VectorSubcoreMesh(
        core_axis_name="c", subcore_axis_name="s",
        num_cores=NC, num_subcores=NS)

    def body(len_hbm, eidx_hbm, packed_hbm, sc13_hbm, out_hbm,
             acc, packed_v, sc13_v,
             len0, len1, len2, idx0, idx1, idx2, sems):
        _sc_body(len_hbm, eidx_hbm, packed_hbm, sc13_hbm, out_hbm,
                 acc, packed_v, sc13_v,
                 (len0, len1, len2), (idx0, idx1, idx2), sems)

    return pl.kernel(
        body,
        out_type=jax.ShapeDtypeStruct((NW * OUT_PAD,), jnp.float32),
        mesh=mesh,
        compiler_params=pltpu.CompilerParams(needs_layout_passes=False),
        scratch_types=[
            pltpu.VMEM((N_NODES,), jnp.float32),
            pltpu.VMEM((PACK_PAD,), jnp.int32),
            pltpu.VMEM((16,), jnp.float32),
            pltpu.VMEM((CHUNK,), jnp.float32),
            pltpu.VMEM((CHUNK,), jnp.float32),
            pltpu.VMEM((CHUNK,), jnp.float32),
            pltpu.VMEM((2, CHUNK), jnp.int32),
            pltpu.VMEM((2, CHUNK), jnp.int32),
            pltpu.VMEM((2, CHUNK), jnp.int32),
            pltpu.SemaphoreType.DMA((NBUF,)),
        ],
    )(edge_length, edge_index, packed, sc13)


def _tc_reduce_body(p_ref, pa_ref, o_ref):
    o_ref[...] = pa_ref[...] + jnp.sum(p_ref[...], axis=0)


@jax.jit
def _tc_reduce(partials, pa_pad):
    return pl.pallas_call(
        _tc_reduce_body,
        grid=(OUT_ROWS // 8,),
        in_specs=[
            pl.BlockSpec((NW, 8, 128), lambda i: (0, i, 0)),
            pl.BlockSpec((8, 128), lambda i: (i, 0)),
        ],
        out_specs=pl.BlockSpec((8, 128), lambda i: (i, 0)),
        out_shape=jax.ShapeDtypeStruct((OUT_ROWS, 128), jnp.float32),
    )(partials, pa_pad)


def kernel(edge_length, edge_index, atom_type, per_atom_energy, per_edge_scales):
    species = atom_type[:, 0].astype(jnp.int32)
    packed = jnp.sum(
        species.reshape(PACK_WORDS, 16) << (2 * jnp.arange(16, dtype=jnp.int32)),
        axis=1, dtype=jnp.int32)
    packed = jnp.pad(packed, (0, PACK_PAD - PACK_WORDS))
    sc13 = (per_edge_scales.astype(jnp.float32) ** 13).reshape(16) / 24.0

    partials = _sc_edge_partials(
        edge_length, edge_index.astype(jnp.int32), packed, sc13)

    pa_pad = jnp.pad(per_atom_energy[:, 0], (0, OUT_PAD - N_NODES)).reshape(
        OUT_ROWS, 128)
    out = _tc_reduce(partials.reshape(NW, OUT_ROWS, 128), pa_pad)
    return out.reshape(OUT_PAD)[:N_NODES, None]


# speedup vs baseline: 1.1070x; 1.1070x over previous
import jax
import jax.numpy as jnp
from jax import lax
from jax.experimental import pallas as pl
from jax.experimental.pallas import tpu as pltpu
from jax.experimental.pallas import tpu_sc as plsc

N_NODES = 100000
N_EDGES = 6400000
R_MAX = 4.0

NC = 2
NS = 16
NW = NC * NS
CHUNK = 2048
N_CHUNKS = N_EDGES // CHUNK
BIG = 21
CPW_HI = 98
NBUF = 3
UNROLL = 8
VEC_ITERS = CHUNK // (16 * UNROLL)

PACK_WORDS = N_NODES // 16
PACK_PAD = 6256

OUT_PAD = 100352
OUT_ROWS = OUT_PAD // 128

CA = 28.0 / 4096.0
CB = 48.0 / 16384.0
CC = 21.0 / 65536.0
RCP_MAGIC = 0x7EF311C3


def _sc_body(len_hbm, eidx_hbm, packed_hbm, sc13_hbm, out_hbm,
             acc, packed_v, sc13_v, len_bufs, idx_bufs, sems):
    cid = lax.axis_index("c")
    sid = lax.axis_index("s")
    wid = sid * NC + cid

    pltpu.sync_copy(packed_hbm, packed_v)
    pltpu.sync_copy(sc13_hbm, sc13_v)

    zeros = jnp.zeros((16,), jnp.float32)

    def zinit(i, _):
        for u in range(10):
            acc[pl.ds(i * 160 + u * 16, 16)] = zeros
        return 0

    lax.fori_loop(0, N_NODES // 160, zinit, 0)

    cnt = jnp.where(wid < BIG, CPW_HI, CPW_HI - 1)
    first = wid * (CPW_HI - 1) + jnp.minimum(wid, BIG)

    def start_chunk(j, buf):
        col = (first + j) * CHUNK
        pltpu.async_copy(len_hbm.at[pl.ds(col, CHUNK)], len_bufs[buf],
                         sems.at[buf])
        pltpu.async_copy(eidx_hbm.at[:, pl.ds(col, CHUNK)], idx_bufs[buf],
                         sems.at[buf])

    def wait_chunk(j, buf):
        col = (first + j) * CHUNK
        pltpu.make_async_copy(len_hbm.at[pl.ds(col, CHUNK)], len_bufs[buf],
                              sems.at[buf]).wait()
        pltpu.make_async_copy(eidx_hbm.at[:, pl.ds(col, CHUNK)],
                              idx_bufs[buf], sems.at[buf]).wait()

    for u in range(NBUF):
        @pl.when(u < cnt)
        def _(u=u):
            start_chunk(u, u)

    def compute_chunk(buf):
        len_v = len_bufs[buf]
        idx_v = idx_bufs[buf]

        @plsc.parallel_loop(0, CHUNK // 16, 1, unroll=UNROLL)
        def vec_body(vi):
            if True:
                off = vi * 16
                ln = len_v[pl.ds(off, 16)]
                c = idx_v[0, pl.ds(off, 16)]
                n = idx_v[1, pl.ds(off, 16)]
                wc = plsc.load_gather(packed_v, [c >> 4])
                wn = plsc.load_gather(packed_v, [n >> 4])
                spc = (wc >> ((c & 15) * 2)) & 3
                spn = (wn >> ((n & 15) * 2)) & 3
                l13 = plsc.load_gather(sc13_v, [spc * 4 + spn])
                y0 = lax.bitcast_convert_type(
                    RCP_MAGIC - lax.bitcast_convert_type(ln, jnp.int32),
                    jnp.float32)
                y1 = y0 * (2.0 - ln * y0)
                y = y1 * (2.0 - ln * y1)
                yy = y * y
                y4 = yy * yy
                y8 = y4 * y4
                inner = y8 - CA * yy + (CB * y - CC)
                eng = jnp.where(ln < R_MAX, y4 * inner * l13, 0.0)
                plsc.addupdate_scatter(acc, [c], eng)

    def tri_body(i, _):
        for u in range(NBUF):
            j = i * NBUF + u

            @pl.when(j < cnt)
            def _(j=j, u=u):
                wait_chunk(j, u)
                compute_chunk(u)

            @pl.when(j + NBUF < cnt)
            def _(j=j, u=u):
                start_chunk(j + NBUF, u)
        return 0

    lax.fori_loop(0, (CPW_HI + NBUF - 1) // NBUF, tri_body, 0)

    pltpu.sync_copy(acc, out_hbm.at[pl.ds(wid * OUT_PAD, N_NODES)])


@jax.jit
def _sc_edge_partials(edge_length, edge_index, packed, sc13):
    mesh = plsc.VectorSubcoreMesh(
        core_axis_name="c", subcore_axis_name="s",
        num_cores=NC, num_subcores=NS)

    def body(len_hbm, eidx_hbm, packed_hbm, sc13_hbm, out_hbm,
             acc, packed_v, sc13_v,
             len0, len1, len2, idx0, idx1, idx2, sems):
        _sc_body(len_hbm, eidx_hbm, packed_hbm, sc13_hbm, out_hbm,
                 acc, packed_v, sc13_v,
                 (len0, len1, len2), (idx0, idx1, idx2), sems)

    return pl.kernel(
        body,
        out_type=jax.ShapeDtypeStruct((NW * OUT_PAD,), jnp.float32),
        mesh=mesh,
        compiler_params=pltpu.CompilerParams(needs_layout_passes=False),
        scratch_types=[
            pltpu.VMEM((N_NODES,), jnp.float32),
            pltpu.VMEM((PACK_PAD,), jnp.int32),
            pltpu.VMEM((16,), jnp.float32),
            pltpu.VMEM((CHUNK,), jnp.float32),
            pltpu.VMEM((CHUNK,), jnp.float32),
            pltpu.VMEM((CHUNK,), jnp.float32),
            pltpu.VMEM((2, CHUNK), jnp.int32),
            pltpu.VMEM((2, CHUNK), jnp.int32),
            pltpu.VMEM((2, CHUNK), jnp.int32),
            pltpu.SemaphoreType.DMA((NBUF,)),
        ],
    )(edge_length, edge_index, packed, sc13)


def _tc_reduce_body(p_ref, pa_ref, o_ref):
    o_ref[...] = pa_ref[...] + jnp.sum(p_ref[...], axis=0)


@jax.jit
def _tc_reduce(partials, pa_pad):
    return pl.pallas_call(
        _tc_reduce_body,
        grid=(OUT_ROWS // 8,),
        in_specs=[
            pl.BlockSpec((NW, 8, 128), lambda i: (0, i, 0)),
            pl.BlockSpec((8, 128), lambda i: (i, 0)),
        ],
        out_specs=pl.BlockSpec((8, 128), lambda i: (i, 0)),
        out_shape=jax.ShapeDtypeStruct((OUT_ROWS, 128), jnp.float32),
    )(partials, pa_pad)


def kernel(edge_length, edge_index, atom_type, per_atom_energy, per_edge_scales):
    species = atom_type[:, 0].astype(jnp.int32)
    packed = jnp.sum(
        species.reshape(PACK_WORDS, 16) << (2 * jnp.arange(16, dtype=jnp.int32)),
        axis=1, dtype=jnp.int32)
    packed = jnp.pad(packed, (0, PACK_PAD - PACK_WORDS))
    sc13 = (per_edge_scales.astype(jnp.float32) ** 13).reshape(16) / 24.0

    partials = _sc_edge_partials(
        edge_length, edge_index.astype(jnp.int32), packed, sc13)

    pa_pad = jnp.pad(per_atom_energy[:, 0], (0, OUT_PAD - N_NODES)).reshape(
        OUT_ROWS, 128)
    out = _tc_reduce(partials.reshape(NW, OUT_ROWS, 128), pa_pad)
    return out.reshape(OUT_PAD)[:N_NODES, None]


# speedup vs baseline: 1.4331x; 1.2946x over previous
import jax
import jax.numpy as jnp
from jax import lax
from jax.experimental import pallas as pl
from jax.experimental.pallas import tpu as pltpu
from jax.experimental.pallas import tpu_sc as plsc

N_NODES = 100000
N_EDGES = 6400000
R_MAX = 4.0

NC = 2
NS = 16
NW = NC * NS
CHUNK = 2048
N_CHUNKS = N_EDGES // CHUNK
BIG = 21
CPW_HI = 98
NBUF = 3
UNROLL = 8
VEC_ITERS = CHUNK // (16 * UNROLL)

PACK_WORDS = N_NODES // 16
PACK_PAD = 6256

OUT_PAD = 100352
OUT_ROWS = OUT_PAD // 128

CA = 28.0 / 4096.0
CB = 48.0 / 16384.0
CC = 21.0 / 65536.0
RCP_MAGIC = 0x7EF311C3


def _sc_body(len_hbm, eidx_hbm, packed_hbm, sc13_hbm, out_hbm,
             acc, packed_v, sc13_v, len_bufs, idx_bufs, sems):
    cid = lax.axis_index("c")
    sid = lax.axis_index("s")
    wid = sid * NC + cid

    pltpu.sync_copy(packed_hbm, packed_v)
    pltpu.sync_copy(sc13_hbm, sc13_v)

    zeros = jnp.zeros((16,), jnp.float32)

    def zinit(i, _):
        for u in range(10):
            acc[pl.ds(i * 160 + u * 16, 16)] = zeros
        return 0

    lax.fori_loop(0, N_NODES // 160, zinit, 0)

    cnt = jnp.where(wid < BIG, CPW_HI, CPW_HI - 1)
    first = wid * (CPW_HI - 1) + jnp.minimum(wid, BIG)

    def start_chunk(j, buf):
        col = (first + j) * CHUNK
        pltpu.async_copy(len_hbm.at[pl.ds(col, CHUNK)], len_bufs[buf],
                         sems.at[buf])
        pltpu.async_copy(eidx_hbm.at[:, pl.ds(col, CHUNK)], idx_bufs[buf],
                         sems.at[buf])

    def wait_chunk(j, buf):
        col = (first + j) * CHUNK
        pltpu.make_async_copy(len_hbm.at[pl.ds(col, CHUNK)], len_bufs[buf],
                              sems.at[buf]).wait()
        pltpu.make_async_copy(eidx_hbm.at[:, pl.ds(col, CHUNK)],
                              idx_bufs[buf], sems.at[buf]).wait()

    for u in range(NBUF):
        @pl.when(u < cnt)
        def _(u=u):
            start_chunk(u, u)

    def compute_chunk(buf):
        len_v = len_bufs[buf]
        idx_v = idx_bufs[buf]

        @plsc.parallel_loop(0, CHUNK // 16, 1, unroll=UNROLL)
        def vec_body(vi):
            if True:
                off = vi * 16
                ln = len_v[pl.ds(off, 16)]
                c = idx_v[0, pl.ds(off, 16)]
                n = idx_v[1, pl.ds(off, 16)]
                wc = plsc.load_gather(packed_v, [c >> 4])
                wn = plsc.load_gather(packed_v, [n >> 4])
                spc = (wc >> ((c & 15) * 2)) & 3
                spn = (wn >> ((n & 15) * 2)) & 3
                l13 = plsc.load_gather(sc13_v, [spc * 4 + spn])
                y0 = lax.bitcast_convert_type(
                    RCP_MAGIC - lax.bitcast_convert_type(ln, jnp.int32),
                    jnp.float32)
                y1 = y0 * (2.0 - ln * y0)
                y = y1 * (2.0 - ln * y1)
                yy = y * y
                y4 = yy * yy
                y8 = y4 * y4
                inner = y8 - CA * yy + (CB * y - CC)
                eng = jnp.where(ln < R_MAX, y4 * inner * l13, 0.0)
                plsc.addupdate_scatter(acc, [c], eng)

    def tri_body(i, _):
        for u in range(NBUF):
            j = i * NBUF + u

            @pl.when(j < cnt)
            def _(j=j, u=u):
                wait_chunk(j, u)
                compute_chunk(u)

            @pl.when(j + NBUF < cnt)
            def _(j=j, u=u):
                start_chunk(j + NBUF, u)
        return 0

    lax.fori_loop(0, (CPW_HI + NBUF - 1) // NBUF, tri_body, 0)

    pltpu.sync_copy(acc, out_hbm.at[pl.ds(wid * OUT_PAD, N_NODES)])


@jax.jit
def _sc_edge_partials(edge_length, edge_index, packed, sc13):
    mesh = plsc.VectorSubcoreMesh(
        core_axis_name="c", subcore_axis_name="s",
        num_cores=NC, num_subcores=NS)

    def body(len_hbm, eidx_hbm, packed_hbm, sc13_hbm, out_hbm,
             acc, packed_v, sc13_v,
             len0, len1, len2, idx0, idx1, idx2, sems):
        _sc_body(len_hbm, eidx_hbm, packed_hbm, sc13_hbm, out_hbm,
                 acc, packed_v, sc13_v,
                 (len0, len1, len2), (idx0, idx1, idx2), sems)

    return pl.kernel(
        body,
        out_type=jax.ShapeDtypeStruct((NW * OUT_PAD,), jnp.float32),
        mesh=mesh,
        compiler_params=pltpu.CompilerParams(needs_layout_passes=False),
        scratch_types=[
            pltpu.VMEM((N_NODES,), jnp.float32),
            pltpu.VMEM((PACK_PAD,), jnp.int32),
            pltpu.VMEM((16,), jnp.float32),
            pltpu.VMEM((CHUNK,), jnp.float32),
            pltpu.VMEM((CHUNK,), jnp.float32),
            pltpu.VMEM((CHUNK,), jnp.float32),
            pltpu.VMEM((2, CHUNK), jnp.int32),
            pltpu.VMEM((2, CHUNK), jnp.int32),
            pltpu.VMEM((2, CHUNK), jnp.int32),
            pltpu.SemaphoreType.DMA((NBUF,)),
        ],
    )(edge_length, edge_index, packed, sc13)


def _tc_reduce_body(p_ref, pa_ref, o_ref):
    o_ref[...] = pa_ref[...] + jnp.sum(p_ref[...], axis=0)


@jax.jit
def _tc_reduce(partials, pa_pad):
    return pl.pallas_call(
        _tc_reduce_body,
        grid=(14,),
        in_specs=[
            pl.BlockSpec((NW, OUT_ROWS // 14, 128), lambda i: (0, i, 0)),
            pl.BlockSpec((OUT_ROWS // 14, 128), lambda i: (i, 0)),
        ],
        out_specs=pl.BlockSpec((OUT_ROWS // 14, 128), lambda i: (i, 0)),
        out_shape=jax.ShapeDtypeStruct((OUT_ROWS, 128), jnp.float32),
    )(partials, pa_pad)


def kernel(edge_length, edge_index, atom_type, per_atom_energy, per_edge_scales):
    species = atom_type[:, 0].astype(jnp.int32)
    packed = jnp.sum(
        species.reshape(PACK_WORDS, 16) << (2 * jnp.arange(16, dtype=jnp.int32)),
        axis=1, dtype=jnp.int32)
    packed = jnp.pad(packed, (0, PACK_PAD - PACK_WORDS))
    sc13 = (per_edge_scales.astype(jnp.float32) ** 13).reshape(16) / 24.0

    partials = _sc_edge_partials(
        edge_length, edge_index.astype(jnp.int32), packed, sc13)

    pa_pad = jnp.pad(per_atom_energy[:, 0], (0, OUT_PAD - N_NODES)).reshape(
        OUT_ROWS, 128)
    out = _tc_reduce(partials.reshape(NW, OUT_ROWS, 128), pa_pad)
    return out.reshape(OUT_PAD)[:N_NODES, None]
